# baseline (speedup 1.0000x reference)
import functools

import jax
import jax.numpy as jnp
import numpy as np
from jax.experimental import pallas as pl

_BATCH = 16384
_NUM_EXPERTS = 64
_NUM_CLASSES = 16
_TOP_K = 8
_IN_DIM = _NUM_EXPERTS * _NUM_CLASSES
_BLOCK = 512

_EXPAND = np.zeros((_NUM_EXPERTS, _IN_DIM), dtype=np.float32)
_EXPAND[np.arange(_IN_DIM) // _NUM_CLASSES, np.arange(_IN_DIM)] = 1.0
_COLLAPSE = np.zeros((_IN_DIM, _NUM_CLASSES), dtype=np.float32)
_COLLAPSE[np.arange(_IN_DIM), np.arange(_IN_DIM) % _NUM_CLASSES] = 1.0


def _moe_block_kernel(x_ref, w1_ref, b1_ref, w2_ref, b2_ref, w3_ref, b3_ref,
                      er_ref, cl_ref, out_ref):
    x = x_ref[...].reshape(_BLOCK, _IN_DIM)
    h = jnp.maximum(
        jnp.dot(x, w1_ref[...], preferred_element_type=jnp.float32)
        + b1_ref[...], 0.0)
    h = jnp.maximum(
        jnp.dot(h, w2_ref[...], preferred_element_type=jnp.float32)
        + b2_ref[...], 0.0)
    logits = (jnp.dot(h, w3_ref[...], preferred_element_type=jnp.float32)
              + b3_ref[...])

    ew = logits
    sel = jnp.zeros(logits.shape, dtype=jnp.bool_)
    m = None
    for _ in range(_TOP_K):
        mx = jnp.max(ew, axis=1, keepdims=True)
        if m is None:
            m = mx
        hit = ew == mx
        sel = jnp.logical_or(sel, hit)
        ew = jnp.where(hit, -jnp.inf, ew)

    w = jnp.where(sel, jnp.exp(logits - m), 0.0)
    w = w / jnp.sum(w, axis=1, keepdims=True)

    w_full = jnp.dot(w, er_ref[...], preferred_element_type=jnp.float32)
    out_ref[...] = jnp.dot(x * w_full, cl_ref[...],
                           preferred_element_type=jnp.float32)


@jax.jit
def kernel(expert_probs, W1, b1, W2, b2, W3, b3):
    B = expert_probs.shape[0]
    grid = (B // _BLOCK,)
    full = lambda shape: pl.BlockSpec(shape, lambda i: (0,) * len(shape))
    return pl.pallas_call(
        _moe_block_kernel,
        grid=grid,
        in_specs=[
            pl.BlockSpec((_BLOCK, _NUM_EXPERTS, _NUM_CLASSES),
                         lambda i: (i, 0, 0)),
            full(W1.shape),
            full(b1.shape),
            full(W2.shape),
            full(b2.shape),
            full(W3.shape),
            full(b3.shape),
            full(_EXPAND.shape),
            full(_COLLAPSE.shape),
        ],
        out_specs=pl.BlockSpec((_BLOCK, _NUM_CLASSES), lambda i: (i, 0)),
        out_shape=jax.ShapeDtypeStruct((B, _NUM_CLASSES), jnp.float32),
    )(expert_probs, W1, b1, W2, b2, W3, b3, jnp.asarray(_EXPAND),
      jnp.asarray(_COLLAPSE))


# speedup vs baseline: 1.1221x; 1.1221x over previous
import functools

import jax
import jax.numpy as jnp
import numpy as np
from jax.experimental import pallas as pl
from jax.experimental.pallas import tpu as pltpu

_BATCH = 16384
_NUM_EXPERTS = 64
_NUM_CLASSES = 16
_TOP_K = 8
_IN_DIM = _NUM_EXPERTS * _NUM_CLASSES
_BLOCK = 512

_EXPAND = np.zeros((_NUM_EXPERTS, _IN_DIM), dtype=np.float32)
_EXPAND[np.arange(_IN_DIM) // _NUM_CLASSES, np.arange(_IN_DIM)] = 1.0
_COLLAPSE = np.zeros((_IN_DIM, _NUM_CLASSES), dtype=np.float32)
_COLLAPSE[np.arange(_IN_DIM), np.arange(_IN_DIM) % _NUM_CLASSES] = 1.0


def _moe_block_kernel(x_ref, w1_ref, b1_ref, w2_ref, b2_ref, w3_ref, b3_ref,
                      er_ref, cl_ref, out_ref):
    x = x_ref[...].reshape(_BLOCK, _IN_DIM)
    h = jnp.maximum(
        jnp.dot(x, w1_ref[...], preferred_element_type=jnp.float32)
        + b1_ref[...], 0.0)
    h = jnp.maximum(
        jnp.dot(h, w2_ref[...], preferred_element_type=jnp.float32)
        + b2_ref[...], 0.0)
    logits = (jnp.dot(h, w3_ref[...], preferred_element_type=jnp.float32)
              + b3_ref[...])

    ew = logits
    sel = jnp.zeros(logits.shape, dtype=jnp.bool_)
    m = None
    for _ in range(_TOP_K):
        mx = jnp.max(ew, axis=1, keepdims=True)
        if m is None:
            m = mx
        hit = ew == mx
        sel = jnp.logical_or(sel, hit)
        ew = jnp.where(hit, -jnp.inf, ew)

    w = jnp.where(sel, jnp.exp(logits - m), 0.0)
    w = w / jnp.sum(w, axis=1, keepdims=True)

    w_full = jnp.dot(w, er_ref[...], preferred_element_type=jnp.float32)
    out_ref[...] = jnp.dot(x * w_full, cl_ref[...],
                           preferred_element_type=jnp.float32)


@jax.jit
def kernel(expert_probs, W1, b1, W2, b2, W3, b3):
    B = expert_probs.shape[0]
    flat8 = expert_probs.reshape(B * 8, 128)
    grid = (B // _BLOCK,)
    full = lambda shape: pl.BlockSpec(shape, lambda i: (0,) * len(shape))
    return pl.pallas_call(
        _moe_block_kernel,
        grid=grid,
        in_specs=[
            pl.BlockSpec((_BLOCK * 8, 128), lambda i: (i, 0)),
            full(W1.shape),
            full(b1.shape),
            full(W2.shape),
            full(b2.shape),
            full(W3.shape),
            full(b3.shape),
            full(_EXPAND.shape),
            full(_COLLAPSE.shape),
        ],
        out_specs=pl.BlockSpec((_BLOCK, _NUM_CLASSES), lambda i: (i, 0)),
        out_shape=jax.ShapeDtypeStruct((B, _NUM_CLASSES), jnp.float32),
    )(flat8, W1, b1, W2, b2, W3, b3, jnp.asarray(_EXPAND),
      jnp.asarray(_COLLAPSE))


# speedup vs baseline: 3.6851x; 3.2843x over previous
import functools

import jax
import jax.numpy as jnp
import numpy as np
from jax.experimental import pallas as pl
from jax.experimental.pallas import tpu as pltpu

_BATCH = 16384
_NUM_EXPERTS = 64
_NUM_CLASSES = 16
_TOP_K = 8
_IN_DIM = _NUM_EXPERTS * _NUM_CLASSES
_BLOCK = 512

_EXPAND = np.zeros((_NUM_EXPERTS, _IN_DIM), dtype=np.float32)
_EXPAND[np.arange(_IN_DIM) // _NUM_CLASSES, np.arange(_IN_DIM)] = 1.0
_COLLAPSE = np.zeros((_IN_DIM, _NUM_CLASSES), dtype=np.float32)
_COLLAPSE[np.arange(_IN_DIM), np.arange(_IN_DIM) % _NUM_CLASSES] = 1.0


def _moe_block_kernel(x_ref, w1_ref, b1_ref, w2_ref, b2_ref, w3_ref, b3_ref,
                      er_ref, cl_ref, out_ref):
    x = x_ref[...].reshape(_BLOCK, _IN_DIM)
    h = jnp.maximum(
        jnp.dot(x, w1_ref[...], preferred_element_type=jnp.float32)
        + b1_ref[...], 0.0)
    h = jnp.maximum(
        jnp.dot(h, w2_ref[...], preferred_element_type=jnp.float32)
        + b2_ref[...], 0.0)
    logits = (jnp.dot(h, w3_ref[...], preferred_element_type=jnp.float32)
              + b3_ref[...])

    ew = logits
    sel = jnp.zeros(logits.shape, dtype=jnp.bool_)
    m = None
    for _ in range(_TOP_K):
        mx = jnp.max(ew, axis=1, keepdims=True)
        if m is None:
            m = mx
        hit = ew == mx
        sel = jnp.logical_or(sel, hit)
        ew = jnp.where(hit, -jnp.inf, ew)

    w = jnp.where(sel, jnp.exp(logits - m), 0.0)
    w = w / jnp.sum(w, axis=1, keepdims=True)

    w_full = jnp.dot(w, er_ref[...], preferred_element_type=jnp.float32)
    out_ref[...] = jnp.dot(x * w_full, cl_ref[...],
                           preferred_element_type=jnp.float32)


@jax.jit
def kernel(expert_probs, W1, b1, W2, b2, W3, b3):
    B = expert_probs.shape[0]
    flat8 = expert_probs.reshape(B, 8, 128)
    grid = (B // _BLOCK,)
    full = lambda shape: pl.BlockSpec(shape, lambda i: (0,) * len(shape))
    return pl.pallas_call(
        _moe_block_kernel,
        grid=grid,
        in_specs=[
            pl.BlockSpec((_BLOCK, 8, 128), lambda i: (i, 0, 0)),
            full(W1.shape),
            full(b1.shape),
            full(W2.shape),
            full(b2.shape),
            full(W3.shape),
            full(b3.shape),
            full(_EXPAND.shape),
            full(_COLLAPSE.shape),
        ],
        out_specs=pl.BlockSpec((_BLOCK, _NUM_CLASSES), lambda i: (i, 0)),
        out_shape=jax.ShapeDtypeStruct((B, _NUM_CLASSES), jnp.float32),
    )(flat8, W1, b1, W2, b2, W3, b3, jnp.asarray(_EXPAND),
      jnp.asarray(_COLLAPSE))


# speedup vs baseline: 4.0263x; 1.0926x over previous
import functools

import jax
import jax.numpy as jnp
import numpy as np
from jax.experimental import pallas as pl
from jax.experimental.pallas import tpu as pltpu

_BATCH = 16384
_NUM_EXPERTS = 64
_NUM_CLASSES = 16
_TOP_K = 8
_IN_DIM = _NUM_EXPERTS * _NUM_CLASSES
_BLOCK = 1024

_EXPAND = np.zeros((_NUM_EXPERTS, _IN_DIM), dtype=np.float32)
_EXPAND[np.arange(_IN_DIM) // _NUM_CLASSES, np.arange(_IN_DIM)] = 1.0
_COLLAPSE = np.zeros((_IN_DIM, _NUM_CLASSES), dtype=np.float32)
_COLLAPSE[np.arange(_IN_DIM), np.arange(_IN_DIM) % _NUM_CLASSES] = 1.0


def _moe_block_kernel(x_ref, w1_ref, b1_ref, w2_ref, b2_ref, w3_ref, b3_ref,
                      er_ref, cl_ref, out_ref):
    x = x_ref[...].reshape(_BLOCK, _IN_DIM)
    h = jnp.maximum(
        jnp.dot(x, w1_ref[...], preferred_element_type=jnp.float32)
        + b1_ref[...], 0.0)
    h = jnp.maximum(
        jnp.dot(h, w2_ref[...], preferred_element_type=jnp.float32)
        + b2_ref[...], 0.0)
    logits = (jnp.dot(h, w3_ref[...], preferred_element_type=jnp.float32)
              + b3_ref[...])

    ew = logits
    sel = jnp.zeros(logits.shape, dtype=jnp.bool_)
    m = None
    for _ in range(_TOP_K):
        mx = jnp.max(ew, axis=1, keepdims=True)
        if m is None:
            m = mx
        hit = ew == mx
        sel = jnp.logical_or(sel, hit)
        ew = jnp.where(hit, -jnp.inf, ew)

    w = jnp.where(sel, jnp.exp(logits - m), 0.0)
    w = w / jnp.sum(w, axis=1, keepdims=True)

    w_full = jnp.dot(w, er_ref[...], preferred_element_type=jnp.float32)
    out_ref[...] = jnp.dot(x * w_full, cl_ref[...],
                           preferred_element_type=jnp.float32)


@jax.jit
def kernel(expert_probs, W1, b1, W2, b2, W3, b3):
    B = expert_probs.shape[0]
    flat8 = expert_probs.reshape(B, 8, 128)
    grid = (B // _BLOCK,)
    full = lambda shape: pl.BlockSpec(shape, lambda i: (0,) * len(shape))
    return pl.pallas_call(
        _moe_block_kernel,
        grid=grid,
        in_specs=[
            pl.BlockSpec((_BLOCK, 8, 128), lambda i: (i, 0, 0)),
            full(W1.shape),
            full(b1.shape),
            full(W2.shape),
            full(b2.shape),
            full(W3.shape),
            full(b3.shape),
            full(_EXPAND.shape),
            full(_COLLAPSE.shape),
        ],
        out_specs=pl.BlockSpec((_BLOCK, _NUM_CLASSES), lambda i: (i, 0)),
        out_shape=jax.ShapeDtypeStruct((B, _NUM_CLASSES), jnp.float32),
    )(flat8, W1, b1, W2, b2, W3, b3, jnp.asarray(_EXPAND),
      jnp.asarray(_COLLAPSE))


# speedup vs baseline: 4.1292x; 1.0256x over previous
import functools

import jax
import jax.numpy as jnp
import numpy as np
from jax.experimental import pallas as pl
from jax.experimental.pallas import tpu as pltpu

_BATCH = 16384
_NUM_EXPERTS = 64
_NUM_CLASSES = 16
_TOP_K = 8
_IN_DIM = _NUM_EXPERTS * _NUM_CLASSES
_BLOCK = 2048

_EXPAND = np.zeros((_NUM_EXPERTS, _IN_DIM), dtype=np.float32)
_EXPAND[np.arange(_IN_DIM) // _NUM_CLASSES, np.arange(_IN_DIM)] = 1.0
_COLLAPSE = np.zeros((_IN_DIM, _NUM_CLASSES), dtype=np.float32)
_COLLAPSE[np.arange(_IN_DIM), np.arange(_IN_DIM) % _NUM_CLASSES] = 1.0


def _moe_block_kernel(x_ref, w1_ref, b1_ref, w2_ref, b2_ref, w3_ref, b3_ref,
                      er_ref, cl_ref, out_ref):
    x = x_ref[...].reshape(_BLOCK, _IN_DIM)
    h = jnp.maximum(
        jnp.dot(x, w1_ref[...], preferred_element_type=jnp.float32)
        + b1_ref[...], 0.0)
    h = jnp.maximum(
        jnp.dot(h, w2_ref[...], preferred_element_type=jnp.float32)
        + b2_ref[...], 0.0)
    logits = (jnp.dot(h, w3_ref[...], preferred_element_type=jnp.float32)
              + b3_ref[...])

    ew = logits
    sel = jnp.zeros(logits.shape, dtype=jnp.bool_)
    m = None
    for _ in range(_TOP_K):
        mx = jnp.max(ew, axis=1, keepdims=True)
        if m is None:
            m = mx
        hit = ew == mx
        sel = jnp.logical_or(sel, hit)
        ew = jnp.where(hit, -jnp.inf, ew)

    w = jnp.where(sel, jnp.exp(logits - m), 0.0)
    w = w / jnp.sum(w, axis=1, keepdims=True)

    w_full = jnp.dot(w, er_ref[...], preferred_element_type=jnp.float32)
    out_ref[...] = jnp.dot(x * w_full, cl_ref[...],
                           preferred_element_type=jnp.float32)


@jax.jit
def kernel(expert_probs, W1, b1, W2, b2, W3, b3):
    B = expert_probs.shape[0]
    flat8 = expert_probs.reshape(B, 8, 128)
    grid = (B // _BLOCK,)
    full = lambda shape: pl.BlockSpec(shape, lambda i: (0,) * len(shape))
    return pl.pallas_call(
        _moe_block_kernel,
        grid=grid,
        in_specs=[
            pl.BlockSpec((_BLOCK, 8, 128), lambda i: (i, 0, 0)),
            full(W1.shape),
            full(b1.shape),
            full(W2.shape),
            full(b2.shape),
            full(W3.shape),
            full(b3.shape),
            full(_EXPAND.shape),
            full(_COLLAPSE.shape),
        ],
        out_specs=pl.BlockSpec((_BLOCK, _NUM_CLASSES), lambda i: (i, 0)),
        out_shape=jax.ShapeDtypeStruct((B, _NUM_CLASSES), jnp.float32),
    )(flat8, W1, b1, W2, b2, W3, b3, jnp.asarray(_EXPAND),
      jnp.asarray(_COLLAPSE))


# speedup vs baseline: 4.1310x; 1.0005x over previous
import functools

import jax
import jax.numpy as jnp
import numpy as np
from jax.experimental import pallas as pl
from jax.experimental.pallas import tpu as pltpu

_BATCH = 16384
_NUM_EXPERTS = 64
_NUM_CLASSES = 16
_TOP_K = 8
_IN_DIM = _NUM_EXPERTS * _NUM_CLASSES
_BLOCK = 2048

_EXPAND = np.zeros((_NUM_EXPERTS, _IN_DIM), dtype=np.float32)
_EXPAND[np.arange(_IN_DIM) // _NUM_CLASSES, np.arange(_IN_DIM)] = 1.0
_COLLAPSE = np.zeros((_IN_DIM, _NUM_CLASSES), dtype=np.float32)
_COLLAPSE[np.arange(_IN_DIM), np.arange(_IN_DIM) % _NUM_CLASSES] = 1.0


def _moe_block_kernel(x_ref, w1_ref, b1_ref, w2_ref, b2_ref, w3_ref, b3_ref,
                      er_ref, cl_ref, out_ref):
    x = x_ref[...].reshape(_BLOCK, _IN_DIM)
    h = jnp.maximum(
        jnp.dot(x.astype(jnp.bfloat16), w1_ref[...].astype(jnp.bfloat16),
                preferred_element_type=jnp.float32)
        + b1_ref[...], 0.0)
    h = jnp.maximum(
        jnp.dot(h.astype(jnp.bfloat16), w2_ref[...].astype(jnp.bfloat16),
                preferred_element_type=jnp.float32)
        + b2_ref[...], 0.0)
    logits = (jnp.dot(h.astype(jnp.bfloat16), w3_ref[...].astype(jnp.bfloat16),
                      preferred_element_type=jnp.float32)
              + b3_ref[...])

    ew = logits
    sel = jnp.zeros(logits.shape, dtype=jnp.bool_)
    m = None
    for _ in range(_TOP_K):
        mx = jnp.max(ew, axis=1, keepdims=True)
        if m is None:
            m = mx
        hit = ew == mx
        sel = jnp.logical_or(sel, hit)
        ew = jnp.where(hit, -jnp.inf, ew)

    w = jnp.where(sel, jnp.exp(logits - m), 0.0)
    w = w / jnp.sum(w, axis=1, keepdims=True)

    w_full = jnp.dot(w, er_ref[...], preferred_element_type=jnp.float32)
    out_ref[...] = jnp.dot(x * w_full, cl_ref[...],
                           preferred_element_type=jnp.float32)


@jax.jit
def kernel(expert_probs, W1, b1, W2, b2, W3, b3):
    B = expert_probs.shape[0]
    flat8 = expert_probs.reshape(B, 8, 128)
    grid = (B // _BLOCK,)
    full = lambda shape: pl.BlockSpec(shape, lambda i: (0,) * len(shape))
    return pl.pallas_call(
        _moe_block_kernel,
        grid=grid,
        in_specs=[
            pl.BlockSpec((_BLOCK, 8, 128), lambda i: (i, 0, 0)),
            full(W1.shape),
            full(b1.shape),
            full(W2.shape),
            full(b2.shape),
            full(W3.shape),
            full(b3.shape),
            full(_EXPAND.shape),
            full(_COLLAPSE.shape),
        ],
        out_specs=pl.BlockSpec((_BLOCK, _NUM_CLASSES), lambda i: (i, 0)),
        out_shape=jax.ShapeDtypeStruct((B, _NUM_CLASSES), jnp.float32),
    )(flat8, W1, b1, W2, b2, W3, b3, jnp.asarray(_EXPAND),
      jnp.asarray(_COLLAPSE))
